# baseline (speedup 1.0000x reference)
import jax
import jax.numpy as jnp
from jax import lax
from jax.experimental import pallas as pl
from jax.experimental.pallas import tpu as pltpu
from jax.experimental.pallas import tpu_sc as plsc

NC, NS = 2, 16
NW = NC * NS
CHUNK = 100
GPC = 2
SUPER = CHUNK * GPC
NBUF = 2
P = 4


def _gather_body(idx_hbm, table_hbm, out_hbm, idxv, bufs, gsem, osems):
    wid = lax.axis_index("s") * NC + lax.axis_index("c")
    nchunks = idx_hbm.shape[1] // GPC
    rows_per_w = nchunks * SUPER
    base0 = wid * rows_per_w

    pltpu.sync_copy(idx_hbm.at[wid], idxv)

    def do_chunk(c, b, drain_first):
        if drain_first:
            pltpu.make_async_copy(
                bufs.at[b], out_hbm.at[pl.ds(base0 + c * SUPER, SUPER)],
                osems[b],
            ).wait()

        cps = [
            pltpu.async_copy(
                table_hbm.at[idxv.at[c * GPC + g]],
                bufs.at[b].at[pl.ds(g * CHUNK, CHUNK)],
                gsem,
            )
            for g in range(GPC)
        ]
        for cp in cps:
            cp.wait()

        pltpu.async_copy(
            bufs.at[b], out_hbm.at[pl.ds(base0 + c * SUPER, SUPER)],
            osems[b])

    for c0 in range(NBUF):
        do_chunk(c0, c0, drain_first=False)

    def step(c, carry):
        for bb in range(NBUF):
            pl.when(lax.rem(c, NBUF) == bb)(
                lambda bb=bb: do_chunk(c, bb, drain_first=True))
        return carry

    lax.fori_loop(NBUF, nchunks, step, 0)

    for b in range(NBUF):
        pltpu.make_async_copy(
            bufs.at[b], out_hbm.at[pl.ds(base0, SUPER)], osems[b]
        ).wait()


def _sc_gather(idx_panels, table):
    nchunks_total = idx_panels.shape[1]
    n = NW * nchunks_total * CHUNK
    d = table.shape[1]
    mesh = plsc.VectorSubcoreMesh(
        core_axis_name="c", subcore_axis_name="s", num_cores=NC, num_subcores=NS
    )
    return pl.kernel(
        _gather_body,
        out_type=jax.ShapeDtypeStruct((n, d), table.dtype),
        mesh=mesh,
        scratch_types=[
            pltpu.VMEM((nchunks_total, CHUNK), jnp.int32),
            pltpu.VMEM((NBUF, SUPER, d), table.dtype),
            pltpu.SemaphoreType.DMA,
            [pltpu.SemaphoreType.DMA] * NBUF,
        ],
    )(idx_panels, table)


def _ln_body(g_ref, s_ref, pos_ref, segt_ref, gam_ref, bet_ref, o_ref):
    s0 = segt_ref[0]
    ds_ = segt_ref[1] - s0
    h = g_ref[...] + pos_ref[...] + s0 + s_ref[...] * ds_
    mean = jnp.mean(h, axis=-1, keepdims=True)
    cent = h - mean
    var = jnp.mean(jnp.square(cent), axis=-1, keepdims=True)
    o_ref[...] = cent * lax.rsqrt(var + 1e-5) * gam_ref[0] + bet_ref[0]


def _tc_ln(g, seg_col, pos_blk, segt, gam, bet, blk):
    n, d = g.shape
    return pl.pallas_call(
        _ln_body,
        grid=(n // blk,),
        in_specs=[
            pl.BlockSpec((blk, d), lambda i: (i, 0)),
            pl.BlockSpec((blk, 1), lambda i: (i, 0)),
            pl.BlockSpec((blk, d), lambda i: (0, 0)),
            pl.BlockSpec((8, d), lambda i: (0, 0)),
            pl.BlockSpec((8, d), lambda i: (0, 0)),
            pl.BlockSpec((8, d), lambda i: (0, 0)),
        ],
        out_specs=pl.BlockSpec((blk, d), lambda i: (i, 0)),
        out_shape=jax.ShapeDtypeStruct((n, d), jnp.float32),
    )(g, seg_col, pos_blk, segt, gam, bet)


def kernel(x, seg, tok_table, pos_table, seg_table, gamma, beta):
    B, L = x.shape
    D = tok_table.shape[1]
    N = B * L
    n_p = N // P
    xf = x.reshape(N).astype(jnp.int32)
    seg_col = seg.reshape(N, 1).astype(jnp.float32)

    BLK = 3200
    pos_blk = jnp.tile(pos_table[:L], (BLK // L, 1))
    segt = jnp.pad(seg_table, ((0, 8 - seg_table.shape[0]), (0, 0)))
    gam = jnp.pad(gamma[None, :], ((0, 7), (0, 0)))
    bet = jnp.pad(beta[None, :], ((0, 7), (0, 0)))

    outs = []
    for p in range(P):
        idx_p = lax.slice(xf, (p * n_p,), ((p + 1) * n_p,))
        idx_panels = idx_p.reshape(NW, n_p // (NW * CHUNK), CHUNK)
        g_p = _sc_gather(idx_panels, tok_table)
        s_p = lax.slice(seg_col, (p * n_p, 0), ((p + 1) * n_p, 1))
        outs.append(_tc_ln(g_p, s_p, pos_blk, segt, gam, bet, BLK))
    out = jnp.concatenate(outs, axis=0)
    return out.reshape(B, L, D)


# speedup vs baseline: 1.5570x; 1.5570x over previous
import jax
import jax.numpy as jnp
from jax import lax
from jax.experimental import pallas as pl
from jax.experimental.pallas import tpu as pltpu
from jax.experimental.pallas import tpu_sc as plsc

NC, NS = 2, 16
NW = NC * NS
CHUNK = 128
GPC = 2
SUPER = CHUNK * GPC
NBUF = 3
P = 1


def _gather_body(idx_hbm, table_hbm, out_hbm, idxv, bufs, gsem, osems):
    wid = lax.axis_index("s") * NC + lax.axis_index("c")
    nchunks = idx_hbm.shape[1] // GPC
    rows_per_w = nchunks * SUPER
    base0 = wid * rows_per_w

    pltpu.sync_copy(idx_hbm.at[wid], idxv)

    def do_chunk(c, b, drain_first):
        if drain_first:
            pltpu.make_async_copy(
                bufs.at[b], out_hbm.at[pl.ds(base0 + c * SUPER, SUPER)],
                osems[b],
            ).wait()

        cps = [
            pltpu.async_copy(
                table_hbm.at[idxv.at[c * GPC + g]],
                bufs.at[b].at[pl.ds(g * CHUNK, CHUNK)],
                gsem,
            )
            for g in range(GPC)
        ]
        for cp in cps:
            cp.wait()

        pltpu.async_copy(
            bufs.at[b], out_hbm.at[pl.ds(base0 + c * SUPER, SUPER)],
            osems[b])

    for c0 in range(NBUF):
        do_chunk(c0, c0, drain_first=False)

    def step(c, carry):
        for bb in range(NBUF):
            pl.when(lax.rem(c, NBUF) == bb)(
                lambda bb=bb: do_chunk(c, bb, drain_first=True))
        return carry

    lax.fori_loop(NBUF, nchunks, step, 0)

    for b in range(NBUF):
        pltpu.make_async_copy(
            bufs.at[b], out_hbm.at[pl.ds(base0, SUPER)], osems[b]
        ).wait()


def _sc_gather(idx_panels, table):
    nchunks_total = idx_panels.shape[1]
    n = NW * nchunks_total * CHUNK
    d = table.shape[1]
    mesh = plsc.VectorSubcoreMesh(
        core_axis_name="c", subcore_axis_name="s", num_cores=NC, num_subcores=NS
    )
    return pl.kernel(
        _gather_body,
        out_type=jax.ShapeDtypeStruct((n, d), table.dtype),
        mesh=mesh,
        scratch_types=[
            pltpu.VMEM((nchunks_total, CHUNK), jnp.int32),
            pltpu.VMEM((NBUF, SUPER, d), table.dtype),
            pltpu.SemaphoreType.DMA,
            [pltpu.SemaphoreType.DMA] * NBUF,
        ],
    )(idx_panels, table)


def _ln_body(g_ref, s_ref, pos_ref, segt_ref, gam_ref, bet_ref, o_ref):
    s0 = segt_ref[0]
    ds_ = segt_ref[1] - s0
    h = g_ref[...] + pos_ref[...] + s0 + s_ref[...] * ds_
    mean = jnp.mean(h, axis=-1, keepdims=True)
    cent = h - mean
    var = jnp.mean(jnp.square(cent), axis=-1, keepdims=True)
    o_ref[...] = cent * lax.rsqrt(var + 1e-5) * gam_ref[0] + bet_ref[0]


def _tc_ln(g, seg_col, pos_blk, segt, gam, bet, blk):
    n, d = g.shape
    return pl.pallas_call(
        _ln_body,
        grid=(n // blk,),
        in_specs=[
            pl.BlockSpec((blk, d), lambda i: (i, 0)),
            pl.BlockSpec((blk, 1), lambda i: (i, 0)),
            pl.BlockSpec((blk, d), lambda i: (0, 0)),
            pl.BlockSpec((8, d), lambda i: (0, 0)),
            pl.BlockSpec((8, d), lambda i: (0, 0)),
            pl.BlockSpec((8, d), lambda i: (0, 0)),
        ],
        out_specs=pl.BlockSpec((blk, d), lambda i: (i, 0)),
        out_shape=jax.ShapeDtypeStruct((n, d), jnp.float32),
    )(g, seg_col, pos_blk, segt, gam, bet)


def kernel(x, seg, tok_table, pos_table, seg_table, gamma, beta):
    B, L = x.shape
    D = tok_table.shape[1]
    N = B * L
    n_p = N // P
    xf = x.reshape(N).astype(jnp.int32)
    seg_col = seg.reshape(N, 1).astype(jnp.float32)

    BLK = 3200
    pos_blk = jnp.tile(pos_table[:L], (BLK // L, 1))
    segt = jnp.pad(seg_table, ((0, 8 - seg_table.shape[0]), (0, 0)))
    gam = jnp.pad(gamma[None, :], ((0, 7), (0, 0)))
    bet = jnp.pad(beta[None, :], ((0, 7), (0, 0)))

    outs = []
    for p in range(P):
        idx_p = lax.slice(xf, (p * n_p,), ((p + 1) * n_p,))
        idx_panels = idx_p.reshape(NW, n_p // (NW * CHUNK), CHUNK)
        g_p = _sc_gather(idx_panels, tok_table)
        s_p = lax.slice(seg_col, (p * n_p, 0), ((p + 1) * n_p, 1))
        outs.append(_tc_ln(g_p, s_p, pos_blk, segt, gam, bet, BLK))
    out = jnp.concatenate(outs, axis=0)
    return out.reshape(B, L, D)
